# Initial kernel scaffold; baseline (speedup 1.0000x reference)
import jax
import jax.numpy as jnp
from jax.experimental import pallas as pl


def kernel(x, batch, num_graphs, W1, a, W2):
    raise NotImplementedError("write your pallas kernel here")


# speedup vs baseline: 6.4746x; 6.4746x over previous
import functools

import jax
import jax.numpy as jnp
from jax.experimental import pallas as pl
from jax.experimental.pallas import tpu as pltpu

N = 100000
C = 256
G = 256
H = 16

BLK = 2000
NBLK = N // BLK


def _pass_a_body(x_ref, b_ref, w1_ref, a_ref, w2_ref, s_ref, seg_acc, cnt_acc):
    i = pl.program_id(0)

    @pl.when(i == 0)
    def _init():
        seg_acc[...] = jnp.zeros_like(seg_acc)
        cnt_acc[...] = jnp.zeros_like(cnt_acc)

    b = b_ref[0, 0, :]
    gi = jax.lax.broadcasted_iota(jnp.int32, (BLK, G), 1)
    oh = jnp.where(gi == b[:, None], 1.0, 0.0).astype(jnp.float32)
    seg_acc[...] += jax.lax.dot_general(
        oh, x_ref[...], (((0,), (0,)), ((), ())),
        preferred_element_type=jnp.float32)
    cnt_acc[...] += jnp.sum(oh, axis=0, keepdims=True)

    @pl.when(i == NBLK - 1)
    def _finish():
        cnt = jnp.maximum(cnt_acc[...], 1.0)
        x_avg = seg_acc[...] / cnt.reshape(G, 1)
        h = jax.lax.dot_general(
            x_avg, w1_ref[...], (((1,), (1,)), ((), ())),
            preferred_element_type=jnp.float32)
        a = a_ref[0]
        h = jnp.where(h >= 0, h, a * h)
        h = jax.lax.dot_general(
            h, w2_ref[...], (((1,), (1,)), ((), ())),
            preferred_element_type=jnp.float32)
        s_ref[...] = jax.nn.sigmoid(h)


def _pass_b_body(x_ref, b_ref, s_ref, o_ref):
    b = b_ref[0, 0, :]
    gi = jax.lax.broadcasted_iota(jnp.int32, (BLK, G), 1)
    oh = jnp.where(gi == b[:, None], 1.0, 0.0).astype(jnp.float32)
    se = jax.lax.dot_general(
        oh, s_ref[...], (((1,), (0,)), ((), ())),
        preferred_element_type=jnp.float32)
    o_ref[...] = x_ref[...] * se


def kernel(x, batch, num_graphs, W1, a, W2):
    bf = jnp.minimum(batch, num_graphs - 1).astype(jnp.int32)
    b3 = bf.reshape(NBLK, 1, BLK)
    a1 = a.reshape(1)

    s = pl.pallas_call(
        _pass_a_body,
        grid=(NBLK,),
        in_specs=[
            pl.BlockSpec((BLK, C), lambda i: (i, 0)),
            pl.BlockSpec((1, 1, BLK), lambda i: (i, 0, 0)),
            pl.BlockSpec((H, C), lambda i: (0, 0)),
            pl.BlockSpec(memory_space=pltpu.SMEM),
            pl.BlockSpec((C, H), lambda i: (0, 0)),
        ],
        out_specs=pl.BlockSpec((G, C), lambda i: (0, 0)),
        out_shape=jax.ShapeDtypeStruct((G, C), jnp.float32),
        scratch_shapes=[
            pltpu.VMEM((G, C), jnp.float32),
            pltpu.VMEM((1, G), jnp.float32),
        ],
    )(x, b3, W1, a1, W2)

    out = pl.pallas_call(
        _pass_b_body,
        grid=(NBLK,),
        in_specs=[
            pl.BlockSpec((BLK, C), lambda i: (i, 0)),
            pl.BlockSpec((1, 1, BLK), lambda i: (i, 0, 0)),
            pl.BlockSpec((G, C), lambda i: (0, 0)),
        ],
        out_specs=pl.BlockSpec((BLK, C), lambda i: (i, 0)),
        out_shape=jax.ShapeDtypeStruct((N, C), jnp.float32),
    )(x, b3, s)
    return out
